# Initial kernel scaffold; baseline (speedup 1.0000x reference)
import jax
import jax.numpy as jnp
from jax.experimental import pallas as pl


def kernel(old_weights, pos, batch, normals, edge_idx_l, dense_l, stddev, W1, b1, W2, b2, W3, b3):
    raise NotImplementedError("write your pallas kernel here")


# speedup vs baseline: 1.0406x; 1.0406x over previous
import functools

import jax
import jax.numpy as jnp
from jax import lax
from jax.experimental import pallas as pl
from jax.experimental.pallas import tpu as pltpu
from jax.experimental.pallas import tpu_sc as plsc

N = 100000
K = 16
H = 32
BN = 512
NPAD = 102400
EPAD = NPAD * K
NROW = NPAD // 128
BROW = 80
CHUNK = 10240
NWORK = 32
PER_W = EPAD // NWORK
SWEEPS = 8


def _sc_gather(table8, idx):
    mesh = plsc.VectorSubcoreMesh(core_axis_name="c", subcore_axis_name="s")

    @functools.partial(
        pl.kernel,
        mesh=mesh,
        out_type=jax.ShapeDtypeStruct((EPAD, 8), jnp.float32),
        scratch_types=[
            pltpu.VMEM((CHUNK,), jnp.int32),
            pltpu.VMEM((CHUNK, 8), jnp.float32),
            pltpu.SemaphoreType.DMA,
        ],
        compiler_params=pltpu.CompilerParams(use_tc_tiling_on_sc=False),
    )
    def gather_k(table_hbm, idx_hbm, out_hbm, idx_v, rows_v, sem):
        wid = lax.axis_index("s") * 2 + lax.axis_index("c")
        for ch in range(PER_W // CHUNK):
            base = wid * PER_W + ch * CHUNK
            pltpu.sync_copy(idx_hbm.at[pl.ds(base, CHUNK)], idx_v)
            pltpu.async_copy(table_hbm.at[idx_v], rows_v, sem).wait()
            pltpu.sync_copy(rows_v, out_hbm.at[pl.ds(base, CHUNK)])

    return gather_k(table8, idx)


def _edge_kernel(std_ref, b3_ref, gath_ref, tab_ref, ow_ref, w1_ref, b1_ref,
                 w2_ref, b2_ref, w3_ref, wout_ref, cov_ref):
    be = BN * K
    g8 = gath_ref[...]
    t8 = tab_ref[...]
    stddev = std_ref[0, 0]

    posn = t8[:, 0:3]
    nrm = t8[:, 3:6]
    posb = jnp.broadcast_to(posn[:, None, :], (BN, K, 3)).reshape(be, 3)
    nrmb = jnp.broadcast_to(nrm[:, None, :], (BN, K, 3)).reshape(be, 3)

    pj = g8[:, 0:3]
    nj = g8[:, 3:6]
    e = pj - posb
    rel = e / stddev
    dist = jnp.sqrt(jnp.sum(rel * rel, axis=1, keepdims=True))
    ndot = jnp.sum(nrmb * nj, axis=1, keepdims=True)
    nproj = jnp.sum(nrmb * rel, axis=1, keepdims=True)
    rowk = lax.broadcasted_iota(jnp.int32, (be, K), 0) % K
    lane = lax.broadcasted_iota(jnp.int32, (be, K), 1)
    onek = (rowk == lane).astype(jnp.float32)
    owb = jnp.broadcast_to(ow_ref[...][:, None, :], (BN, K, K)).reshape(be, K)
    ow = jnp.sum(owb * onek, axis=1, keepdims=True)
    zero = jnp.zeros((be, 1), jnp.float32)
    feat = jnp.concatenate([rel, dist, ndot, nproj, ow, zero], axis=1)

    h = jnp.dot(feat, w1_ref[...], preferred_element_type=jnp.float32)
    h = jnp.maximum(h + b1_ref[...], 0.0)
    hr = h.reshape(BN, K, H)
    m = hr[:, 0, :]
    for k in range(1, K):
        m = m + hr[:, k, :]
    mb = jnp.broadcast_to(m[:, None, :], (BN, K, H)).reshape(be, H)
    cat = jnp.concatenate([h, mb], axis=1)
    g2 = jnp.dot(cat, w2_ref[...], preferred_element_type=jnp.float32)
    g2 = jnp.maximum(g2 + b2_ref[...], 0.0)

    def rbf(x):
        return x.astype(jnp.bfloat16).astype(jnp.float32)

    logits = jnp.sum(rbf(g2) * rbf(w3_ref[...]), axis=1, keepdims=True) \
        + b3_ref[0, 0]
    w = 1.0 / (1.0 + jnp.exp(-logits))

    def to_nk(col):
        spread = col * onek
        return jnp.sum(spread.reshape(BN, K, K), axis=1)

    wk = to_nk(w)
    wout_ref[...] = wk

    ex, ey, ez = e[:, 0:1], e[:, 1:2], e[:, 2:3]
    wx, wy, wz = rbf(w * ex), rbf(w * ey), rbf(w * ez)
    ex, ey, ez = rbf(ex), rbf(ey), rbf(ez)

    def ktree(t):
        x = t
        for _ in range(4):
            half = x.shape[0] // 2
            x = jnp.sum(x.reshape(half, 2, 1), axis=1)
        return x

    c00 = ktree(wx * ex)
    c01 = ktree(wx * ey)
    c02 = ktree(wx * ez)
    c11 = ktree(wy * ey)
    c12 = ktree(wy * ez)
    c22 = ktree(wz * ez)
    zn = jnp.zeros((BN, 1), jnp.float32)
    cov_ref[...] = jnp.concatenate([c00, c01, c02, c11, c12, c22, zn, zn],
                                   axis=1)


def _edge_pass(gath, table8, owp, stddev2, w1p, b1r, w2, b2r, w3r, b3s):
    grid = NPAD // BN
    be = BN * K
    return pl.pallas_call(
        _edge_kernel,
        grid=(grid,),
        in_specs=[
            pl.BlockSpec(memory_space=pltpu.SMEM),
            pl.BlockSpec(memory_space=pltpu.SMEM),
            pl.BlockSpec((be, 8), lambda i: (i, 0)),
            pl.BlockSpec((BN, 8), lambda i: (i, 0)),
            pl.BlockSpec((BN, K), lambda i: (i, 0)),
            pl.BlockSpec((8, H), lambda i: (0, 0)),
            pl.BlockSpec((1, H), lambda i: (0, 0)),
            pl.BlockSpec((2 * H, H), lambda i: (0, 0)),
            pl.BlockSpec((1, H), lambda i: (0, 0)),
            pl.BlockSpec((1, H), lambda i: (0, 0)),
        ],
        out_specs=[
            pl.BlockSpec((BN, K), lambda i: (i, 0)),
            pl.BlockSpec((BN, 8), lambda i: (i, 0)),
        ],
        out_shape=[
            jax.ShapeDtypeStruct((NPAD, K), jnp.float32),
            jax.ShapeDtypeStruct((NPAD, 8), jnp.float32),
        ],
        compiler_params=pltpu.CompilerParams(
            dimension_semantics=("arbitrary",),
        ),
    )(stddev2, b3s, gath, table8, owp, w1p, b1r, w2, b2r, w3r)


def _rot(app, apq, aqq):
    bsafe = jnp.where(apq == 0.0, 1.0, apq)
    zeta = (app - aqq) / (2.0 * bsafe)
    t = jnp.sign(zeta) / (jnp.abs(zeta) + jnp.sqrt(1.0 + zeta * zeta))
    t = jnp.where(apq == 0.0, 0.0, t)
    c = lax.rsqrt(1.0 + t * t)
    return c, t * c


_SCHED = (((0, 2), (1, 3)), ((0, 3), (2, 1)), ((0, 1), (3, 2)))


def _eigh_kernel(cov_ref, out_ref):
    sh = (BROW, 128)
    zero = jnp.zeros(sh, jnp.float32)
    one = jnp.ones(sh, jnp.float32)
    A = [[zero for _ in range(4)] for _ in range(4)]
    A[0][0] = cov_ref[0]
    A[0][1] = A[1][0] = cov_ref[1]
    A[0][2] = A[2][0] = cov_ref[2]
    A[1][1] = cov_ref[3]
    A[1][2] = A[2][1] = cov_ref[4]
    A[2][2] = cov_ref[5]
    V = [[one if i == j else zero for j in range(4)] for i in range(4)]

    for _ in range(SWEEPS):
        for pairs in _SCHED:
            cs = [_rot(A[p][p], A[p][q], A[q][q]) for (p, q) in pairs]
            for (p, q), (c, s) in zip(pairs, cs):
                for j in range(4):
                    rp = c * A[p][j] + s * A[q][j]
                    rq = c * A[q][j] - s * A[p][j]
                    A[p][j], A[q][j] = rp, rq
            for (p, q), (c, s) in zip(pairs, cs):
                for i in range(4):
                    cp = c * A[i][p] + s * A[i][q]
                    cq = c * A[i][q] - s * A[i][p]
                    A[i][p], A[i][q] = cp, cq
                    vp = c * V[i][p] + s * V[i][q]
                    vq = c * V[i][q] - s * V[i][p]
                    V[i][p], V[i][q] = vp, vq

    lam = [A[0][0], A[1][1], A[2][2]]
    cols = [[V[0][j], V[1][j], V[2][j]] for j in range(3)]

    def cswap(i, j):
        swap = lam[i] > lam[j]
        lam[i], lam[j] = (jnp.where(swap, lam[j], lam[i]),
                          jnp.where(swap, lam[i], lam[j]))
        for r in range(3):
            a, b = cols[i][r], cols[j][r]
            cols[i][r] = jnp.where(swap, b, a)
            cols[j][r] = jnp.where(swap, a, b)

    cswap(0, 1)
    cswap(1, 2)
    cswap(0, 1)

    a0, a1, a2 = jnp.abs(lam[0]), jnp.abs(lam[1]), jnp.abs(lam[2])
    take1 = a1 < a0
    best01 = jnp.where(take1, a1, a0)
    take2 = a2 < best01
    for r in range(3):
        v01 = jnp.where(take1, cols[1][r], cols[0][r])
        out_ref[r] = jnp.where(take2, cols[2][r], v01)


def _eigh_pass(covt):
    grid = NROW // BROW
    return pl.pallas_call(
        _eigh_kernel,
        grid=(grid,),
        in_specs=[pl.BlockSpec((8, BROW, 128), lambda i: (0, i, 0))],
        out_specs=pl.BlockSpec((3, BROW, 128), lambda i: (0, i, 0)),
        out_shape=jax.ShapeDtypeStruct((3, NROW, 128), jnp.float32),
        compiler_params=pltpu.CompilerParams(
            dimension_semantics=("arbitrary",),
        ),
    )(covt)


def kernel(old_weights, pos, batch, normals, edge_idx_l, dense_l, stddev,
           W1, b1, W2, b2, W3, b3):
    table8 = jnp.pad(jnp.concatenate([pos, normals], axis=1),
                     ((0, NPAD - N), (0, 2)))
    idx = jnp.pad(dense_l.reshape(-1), (0, EPAD - N * K))
    owp = jnp.pad(old_weights, ((0, NPAD - N), (0, 0)))
    w1p = jnp.pad(W1, ((0, 1), (0, 0)))
    b1r = b1.reshape(1, H)
    b2r = b2.reshape(1, H)
    w3r = W3.reshape(1, H)
    b3s = b3.reshape(1, 1)
    stddev2 = stddev.reshape(1, 1)

    gath = _sc_gather(table8, idx)
    weights_pad, cov8 = _edge_pass(gath, table8, owp, stddev2, w1p, b1r, w2=W2,
                                   b2r=b2r, w3r=w3r, b3s=b3s)
    del cov8
    weights_n = weights_pad[:N]
    edge_pos = pos[dense_l] - pos[:, None, :]
    wep = weights_n[:, :, None] * edge_pos
    cov = jnp.einsum('nki,nkj->nij', wep, edge_pos)
    eig_val, eig_vec = jnp.linalg.eigh(cov)
    order = jnp.argsort(jnp.abs(eig_val), axis=-1)
    eig_vec = jnp.take_along_axis(eig_vec, order[:, None, :], axis=2)
    new_normals = eig_vec[:, :, 0]
    weights = weights_pad[:N]
    return (new_normals, weights)


# speedup vs baseline: 1.0817x; 1.0394x over previous
import functools

import jax
import jax.numpy as jnp
from jax import lax
from jax.experimental import pallas as pl
from jax.experimental.pallas import tpu as pltpu
from jax.experimental.pallas import tpu_sc as plsc

N = 100000
K = 16
H = 32
BN = 512
NPAD = 102400
EPAD = NPAD * K
NROW = NPAD // 128
BROW = 80
CHUNK = 10240
NWORK = 32
PER_W = EPAD // NWORK
SWEEPS = 8


def _sc_gather(table8, idx):
    mesh = plsc.VectorSubcoreMesh(core_axis_name="c", subcore_axis_name="s")

    @functools.partial(
        pl.kernel,
        mesh=mesh,
        out_type=jax.ShapeDtypeStruct((EPAD, 8), jnp.float32),
        scratch_types=[
            pltpu.VMEM((CHUNK,), jnp.int32),
            pltpu.VMEM((CHUNK, 8), jnp.float32),
            pltpu.SemaphoreType.DMA,
        ],
        compiler_params=pltpu.CompilerParams(use_tc_tiling_on_sc=False),
    )
    def gather_k(table_hbm, idx_hbm, out_hbm, idx_v, rows_v, sem):
        wid = lax.axis_index("s") * 2 + lax.axis_index("c")
        for ch in range(PER_W // CHUNK):
            base = wid * PER_W + ch * CHUNK
            pltpu.sync_copy(idx_hbm.at[pl.ds(base, CHUNK)], idx_v)
            pltpu.async_copy(table_hbm.at[idx_v], rows_v, sem).wait()
            pltpu.sync_copy(rows_v, out_hbm.at[pl.ds(base, CHUNK)])

    return gather_k(table8, idx)


def _edge_kernel(std_ref, b3_ref, gath_ref, tab_ref, ow_ref, w1_ref, b1_ref,
                 w2_ref, b2_ref, w3_ref, wout_ref):
    be = BN * K
    g8 = gath_ref[...]
    t8 = tab_ref[...]
    stddev = std_ref[0, 0]

    posn = t8[:, 0:3]
    nrm = t8[:, 3:6]
    posb = jnp.broadcast_to(posn[:, None, :], (BN, K, 3)).reshape(be, 3)
    nrmb = jnp.broadcast_to(nrm[:, None, :], (BN, K, 3)).reshape(be, 3)

    pj = g8[:, 0:3]
    nj = g8[:, 3:6]
    e = pj - posb
    rel = e / stddev
    dist = jnp.sqrt(jnp.sum(rel * rel, axis=1, keepdims=True))
    ndot = jnp.sum(nrmb * nj, axis=1, keepdims=True)
    nproj = jnp.sum(nrmb * rel, axis=1, keepdims=True)
    rowk = lax.broadcasted_iota(jnp.int32, (be, K), 0) % K
    lane = lax.broadcasted_iota(jnp.int32, (be, K), 1)
    onek = (rowk == lane).astype(jnp.float32)
    owb = jnp.broadcast_to(ow_ref[...][:, None, :], (BN, K, K)).reshape(be, K)
    ow = jnp.sum(owb * onek, axis=1, keepdims=True)
    zero = jnp.zeros((be, 1), jnp.float32)
    feat = jnp.concatenate([rel, dist, ndot, nproj, ow, zero], axis=1)

    h = jnp.dot(feat, w1_ref[...], preferred_element_type=jnp.float32)
    h = jnp.maximum(h + b1_ref[...], 0.0)
    hr = h.reshape(BN, K, H)
    m = hr[:, 0, :]
    for k in range(1, K):
        m = m + hr[:, k, :]
    mb = jnp.broadcast_to(m[:, None, :], (BN, K, H)).reshape(be, H)
    cat = jnp.concatenate([h, mb], axis=1)
    g2 = jnp.dot(cat, w2_ref[...], preferred_element_type=jnp.float32)
    g2 = jnp.maximum(g2 + b2_ref[...], 0.0)

    def rbf(x):
        return x.astype(jnp.bfloat16).astype(jnp.float32)

    logits = jnp.sum(rbf(g2) * rbf(w3_ref[...]), axis=1, keepdims=True) \
        + b3_ref[0, 0]
    w = 1.0 / (1.0 + jnp.exp(-logits))

    def to_nk(col):
        spread = col * onek
        return jnp.sum(spread.reshape(BN, K, K), axis=1)

    wk = to_nk(w)
    wout_ref[...] = wk


def _edge_pass(gath, table8, owp, stddev2, w1p, b1r, w2, b2r, w3r, b3s):
    grid = NPAD // BN
    be = BN * K
    return pl.pallas_call(
        _edge_kernel,
        grid=(grid,),
        in_specs=[
            pl.BlockSpec(memory_space=pltpu.SMEM),
            pl.BlockSpec(memory_space=pltpu.SMEM),
            pl.BlockSpec((be, 8), lambda i: (i, 0)),
            pl.BlockSpec((BN, 8), lambda i: (i, 0)),
            pl.BlockSpec((BN, K), lambda i: (i, 0)),
            pl.BlockSpec((8, H), lambda i: (0, 0)),
            pl.BlockSpec((1, H), lambda i: (0, 0)),
            pl.BlockSpec((2 * H, H), lambda i: (0, 0)),
            pl.BlockSpec((1, H), lambda i: (0, 0)),
            pl.BlockSpec((1, H), lambda i: (0, 0)),
        ],
        out_specs=pl.BlockSpec((BN, K), lambda i: (i, 0)),
        out_shape=jax.ShapeDtypeStruct((NPAD, K), jnp.float32),
        compiler_params=pltpu.CompilerParams(
            dimension_semantics=("arbitrary",),
        ),
    )(stddev2, b3s, gath, table8, owp, w1p, b1r, w2, b2r, w3r)


def kernel(old_weights, pos, batch, normals, edge_idx_l, dense_l, stddev,
           W1, b1, W2, b2, W3, b3):
    table8 = jnp.pad(jnp.concatenate([pos, normals], axis=1),
                     ((0, NPAD - N), (0, 2)))
    idx = jnp.pad(dense_l.reshape(-1), (0, EPAD - N * K))
    owp = jnp.pad(old_weights, ((0, NPAD - N), (0, 0)))
    w1p = jnp.pad(W1, ((0, 1), (0, 0)))
    b1r = b1.reshape(1, H)
    b2r = b2.reshape(1, H)
    w3r = W3.reshape(1, H)
    b3s = b3.reshape(1, 1)
    stddev2 = stddev.reshape(1, 1)

    gath = _sc_gather(table8, idx)
    weights_pad = _edge_pass(gath, table8, owp, stddev2, w1p, b1r, w2=W2,
                             b2r=b2r, w3r=w3r, b3s=b3s)
    weights_n = weights_pad[:N]
    edge_pos = pos[dense_l] - pos[:, None, :]
    wep = weights_n[:, :, None] * edge_pos
    cov = jnp.einsum('nki,nkj->nij', wep, edge_pos)
    eig_val, eig_vec = jnp.linalg.eigh(cov)
    order = jnp.argsort(jnp.abs(eig_val), axis=-1)
    eig_vec = jnp.take_along_axis(eig_vec, order[:, None, :], axis=2)
    new_normals = eig_vec[:, :, 0]
    weights = weights_pad[:N]
    return (new_normals, weights)
